# baseline (speedup 1.0000x reference)
import functools
import math

import jax
import jax.numpy as jnp
from jax import lax
from jax.experimental import pallas as pl
from jax.experimental.pallas import tpu as pltpu

_B, _N, _C = 2, 20000, 91
_C1 = _C - 1
_NC = _N * _C1
_SCORE_THRESH = 0.05
_NMS_THRESH = 0.5
_DET = 100
_TOPK = 1000
_M = 1024
_CH = 128
_NCHUNK = _M // _CH
_IMG_W = 800.0
_IMG_H = 800.0
_CLIP = math.log(1000.0 / 16.0)
_NEG = -1e9

_R = 1000


_KCH = 20
_SB = 8
_LN = (_B * _N) // (_KCH * _SB)


def _k1_body(logits_ref, d_ref, prop_ref, rmax_ref):
    l = logits_ref[:, 0]
    m = jnp.max(l, axis=0, keepdims=True)
    e = jnp.exp(l - m)
    s = jnp.sum(e, axis=0, keepdims=True)
    scores = (e / s)[1:]

    d = d_ref[:, 0].reshape(_C, 4, _SB, _LN)
    dx = d[1:, 0] / 10.0
    dy = d[1:, 1] / 10.0
    dw = jnp.minimum(d[1:, 2] / 5.0, _CLIP)
    dh = jnp.minimum(d[1:, 3] / 5.0, _CLIP)

    p = prop_ref[:, 0]
    px1 = p[0:1]
    py1 = p[1:2]
    px2 = p[2:3]
    py2 = p[3:4]
    w = px2 - px1
    h = py2 - py1
    cx = px1 + 0.5 * w
    cy = py1 + 0.5 * h

    pcx = dx * w + cx
    pcy = dy * h + cy
    pw = jnp.exp(dw) * w
    ph = jnp.exp(dh) * h
    x1 = jnp.clip(pcx - 0.5 * pw, 0.0, _IMG_W)
    y1 = jnp.clip(pcy - 0.5 * ph, 0.0, _IMG_H)
    x2 = jnp.clip(pcx + 0.5 * pw, 0.0, _IMG_W)
    y2 = jnp.clip(pcy + 0.5 * ph, 0.0, _IMG_H)

    valid = (scores > _SCORE_THRESH) & ((x2 - x1) >= 0.01) & ((y2 - y1) >= 0.01)
    masked = jnp.where(valid, scores, _NEG)
    rmax_ref[0] = jnp.max(masked, axis=0)


def _row_scores(logitsT, regT, propsT, interpret=False):
    return pl.pallas_call(
        _k1_body,
        grid=(_KCH,),
        in_specs=[
            pl.BlockSpec((_C, 1, _SB, _LN), lambda i: (0, i, 0, 0)),
            pl.BlockSpec((4 * _C, 1, _SB, _LN), lambda i: (0, i, 0, 0)),
            pl.BlockSpec((4, 1, _SB, _LN), lambda i: (0, i, 0, 0)),
        ],
        out_specs=[
            pl.BlockSpec((1, _SB, _LN), lambda i: (i, 0, 0)),
        ],
        out_shape=[
            jax.ShapeDtypeStruct((_KCH, _SB, _LN), jnp.float32),
        ],
        interpret=interpret,
    )(logitsT, regT, propsT)


def _k2_body(rows_ref, l_ref, r_ref, p_ref, sub_ref, regr_ref, propr_ref,
             lacc_ref):
    i = pl.program_id(1)
    rows = rows_ref[0, 0]
    hp = lax.Precision.HIGHEST
    dn = (((1,), (1,)), ((), ()))
    l3 = l_ref[:, 0]
    r3 = r_ref[:, 0]
    p3 = p_ref[:, 0]
    ln = lax.broadcasted_iota(jnp.int32, (_M, _LN), 1)
    logc = jnp.zeros((_M, _C), jnp.float32)
    regc = jnp.zeros((_M, 4 * _C), jnp.float32)
    propc = jnp.zeros((_M, 4), jnp.float32)
    for k in range(_SB):
        colid = i * (_SB * _LN) + k * _LN + ln
        ohk = (rows[:, None] == colid).astype(jnp.float32)
        logc = logc + lax.dot_general(ohk, l3[:, k, :], dn, precision=hp)
        regc = regc + lax.dot_general(ohk, r3[:, k, :], dn, precision=hp)
        propc = propc + lax.dot_general(ohk, p3[:, k, :], dn, precision=hp)

    @pl.when(i == 0)
    def _():
        lacc_ref[...] = logc
        regr_ref[0] = regc
        propr_ref[0] = propc

    @pl.when(i > 0)
    def _():
        lacc_ref[...] += logc
        regr_ref[0] += regc
        propr_ref[0] += propc

    @pl.when(i == _KCH - 1)
    def _():
        l = lacc_ref[...]
        m = jnp.max(l, axis=-1, keepdims=True)
        e = jnp.exp(l - m)
        s = jnp.sum(e, axis=-1, keepdims=True)
        scores = (e / s)[:, 1:]

        d = regr_ref[0]
        ii2 = lax.broadcasted_iota(jnp.int32, (4 * _C, _C), 0)
        jj2 = lax.broadcasted_iota(jnp.int32, (4 * _C, _C), 1)

        def comp(k):
            sm = (ii2 == 4 * jj2 + k).astype(jnp.float32)
            return lax.dot(d, sm, precision=hp)[:, 1:]

        dx = comp(0) / 10.0
        dy = comp(1) / 10.0
        dw = jnp.minimum(comp(2) / 5.0, _CLIP)
        dh = jnp.minimum(comp(3) / 5.0, _CLIP)

        p = propr_ref[0]
        px1 = p[:, 0:1]
        py1 = p[:, 1:2]
        px2 = p[:, 2:3]
        py2 = p[:, 3:4]
        w = px2 - px1
        h = py2 - py1
        cx = px1 + 0.5 * w
        cy = py1 + 0.5 * h
        pcx = dx * w + cx
        pcy = dy * h + cy
        pw = jnp.exp(dw) * w
        ph = jnp.exp(dh) * h
        x1 = jnp.clip(pcx - 0.5 * pw, 0.0, _IMG_W)
        y1 = jnp.clip(pcy - 0.5 * ph, 0.0, _IMG_H)
        x2 = jnp.clip(pcx + 0.5 * pw, 0.0, _IMG_W)
        y2 = jnp.clip(pcy + 0.5 * ph, 0.0, _IMG_H)
        valid = ((scores > _SCORE_THRESH) & ((x2 - x1) >= 0.01)
                 & ((y2 - y1) >= 0.01))
        sub_ref[0] = jnp.where(valid, scores, _NEG)


def _gather_rows(rows_global, logitsT, regT, propsT, interpret=False):
    return pl.pallas_call(
        _k2_body,
        grid=(_B, _KCH),
        in_specs=[
            pl.BlockSpec((1, 1, _M), lambda b, i: (b, 0, 0)),
            pl.BlockSpec((_C, 1, _SB, _LN), lambda b, i: (0, i, 0, 0)),
            pl.BlockSpec((4 * _C, 1, _SB, _LN), lambda b, i: (0, i, 0, 0)),
            pl.BlockSpec((4, 1, _SB, _LN), lambda b, i: (0, i, 0, 0)),
        ],
        out_specs=[
            pl.BlockSpec((1, _M, _C1), lambda b, i: (b, 0, 0)),
            pl.BlockSpec((1, _M, 4 * _C), lambda b, i: (b, 0, 0)),
            pl.BlockSpec((1, _M, 4), lambda b, i: (b, 0, 0)),
        ],
        out_shape=[
            jax.ShapeDtypeStruct((_B, _M, _C1), jnp.float32),
            jax.ShapeDtypeStruct((_B, _M, 4 * _C), jnp.float32),
            jax.ShapeDtypeStruct((_B, _M, 4), jnp.float32),
        ],
        scratch_shapes=[pltpu.VMEM((_M, _C), jnp.float32)],
        interpret=interpret,
    )(rows_global[:, None, :], logitsT, regT, propsT)


def _k3_body(sc_ref, idx_ref, slot_ref, regr_ref, propr_ref,
             ob_ref, os_ref, ol_ref, adj_ref):
    s = sc_ref[0, 0]
    idx = idx_ref[0, 0]
    idxf = idx.astype(jnp.float32)
    slot = slot_ref[0, 0]

    hp = lax.Precision.HIGHEST
    slots = lax.broadcasted_iota(jnp.int32, (_M, _M), 1)
    ohr = (slot[:, None] == slots).astype(jnp.float32)
    rsel = lax.dot(ohr, regr_ref[0], precision=hp)
    psel = lax.dot(ohr, propr_ref[0], precision=hp)

    rparts = []
    for c in range(_NCHUNK):
        sl = slice(c * _CH, (c + 1) * _CH)
        src = s[sl][:, None]
        idc = idxf[sl][:, None]
        gt = (s[None, :] > src) | ((s[None, :] == src) & (idxf[None, :] < idc))
        rparts.append(jnp.sum(gt.astype(jnp.int32), axis=1))
    r = jnp.concatenate(rparts)
    top = r < _TOPK

    cls = idx % _C1 + 1

    ii = lax.broadcasted_iota(jnp.int32, (4 * _C, _C), 0)
    jj = lax.broadcasted_iota(jnp.int32, (4 * _C, _C), 1)
    clsoh = (cls[:, None] == lax.broadcasted_iota(jnp.int32, (_M, _C), 1)
             ).astype(jnp.float32)

    def rcomp(k):
        sm = (ii == 4 * jj + k).astype(jnp.float32)
        tmp = lax.dot(rsel, sm, precision=hp)
        return jnp.sum(tmp * clsoh, axis=1)

    px1 = psel[:, 0]
    py1 = psel[:, 1]
    px2 = psel[:, 2]
    py2 = psel[:, 3]
    w = px2 - px1
    h = py2 - py1
    cx = px1 + 0.5 * w
    cy = py1 + 0.5 * h
    dx = rcomp(0) / 10.0
    dy = rcomp(1) / 10.0
    dw = jnp.minimum(rcomp(2) / 5.0, _CLIP)
    dh = jnp.minimum(rcomp(3) / 5.0, _CLIP)
    pcx = dx * w + cx
    pcy = dy * h + cy
    pw = jnp.exp(dw) * w
    ph = jnp.exp(dh) * h
    x1 = jnp.clip(pcx - 0.5 * pw, 0.0, _IMG_W)
    y1 = jnp.clip(pcy - 0.5 * ph, 0.0, _IMG_H)
    x2 = jnp.clip(pcx + 0.5 * pw, 0.0, _IMG_W)
    y2 = jnp.clip(pcy + 0.5 * ph, 0.0, _IMG_H)

    offs = cls.astype(jnp.float32) * 4096.0
    bx1 = x1 + offs
    by1 = y1 + offs
    bx2 = x2 + offs
    by2 = y2 + offs
    area = (bx2 - bx1) * (by2 - by1)

    topf = top.astype(jnp.float32)
    for c in range(_NCHUNK):
        sl = slice(c * _CH, (c + 1) * _CH)
        ltx = jnp.maximum(bx1[sl][:, None], bx1[None, :])
        lty = jnp.maximum(by1[sl][:, None], by1[None, :])
        rbx = jnp.minimum(bx2[sl][:, None], bx2[None, :])
        rby = jnp.minimum(by2[sl][:, None], by2[None, :])
        inter = jnp.clip(rbx - ltx, 0.0, None) * jnp.clip(rby - lty, 0.0, None)
        union = area[sl][:, None] + area[None, :] - inter
        iou = inter / jnp.maximum(union, 1e-9)
        adjc = ((iou > _NMS_THRESH) & (r[sl][:, None] < r[None, :])).astype(jnp.float32)
        adj_ref[sl, :] = adjc * topf[sl][:, None] * topf[None, :]

    def cond(carry):
        _, changed, it = carry
        return (changed > 0) & (it < _M + 1)

    def body(carry):
        keepf, _, it = carry
        sup = jnp.zeros((_M,), jnp.float32)
        for c in range(_NCHUNK):
            sl = slice(c * _CH, (c + 1) * _CH)
            sup = jnp.maximum(sup, jnp.max(keepf[sl][:, None] * adj_ref[sl, :], axis=0))
        nk = topf * (1.0 - jnp.minimum(sup, 1.0))
        ch = jnp.sum(jnp.abs(nk - keepf)).astype(jnp.int32)
        return nk, ch, it + 1

    keepf, _, _ = lax.while_loop(cond, body, (topf, jnp.int32(1), jnp.int32(0)))

    condf = keepf * (s > -1e8).astype(jnp.float32)
    cond_b = condf > 0
    uf = topf * (1.0 - condf)
    kb = jnp.zeros((_M,), jnp.float32)
    ub = jnp.zeros((_M,), jnp.float32)
    for c in range(_NCHUNK):
        sl = slice(c * _CH, (c + 1) * _CH)
        rlt = (r[sl][:, None] < r[None, :]).astype(jnp.float32)
        kb = kb + jnp.sum(condf[sl][:, None] * rlt, axis=0)
        ub = ub + jnp.sum(uf[sl][:, None] * rlt, axis=0)
    nktot = jnp.sum(condf)
    frank = jnp.where(cond_b, kb, nktot + ub).astype(jnp.int32)
    frank = jnp.where(top, frank, jnp.int32(100000))

    krow = lax.broadcasted_iota(jnp.int32, (128, _M), 0)
    oh2 = (frank[None, :] == krow).astype(jnp.float32)

    def sel(x):
        return jnp.sum(oh2 * x[None, :], axis=1)

    fsc = jnp.where(cond_b, s, _NEG)
    os_ref[0, 0] = sel(fsc)
    ol_ref[0, 0] = sel(cls.astype(jnp.float32)).astype(jnp.int32)
    ob_ref[0] = jnp.stack([sel(x1), sel(y1), sel(x2), sel(y2)], axis=0)


def _nms_kernel(sc, idx, slot, regrows, proprows, interpret=False):
    return pl.pallas_call(
        _k3_body,
        grid=(_B,),
        in_specs=[
            pl.BlockSpec((1, 1, _M), lambda b: (b, 0, 0)),
            pl.BlockSpec((1, 1, _M), lambda b: (b, 0, 0)),
            pl.BlockSpec((1, 1, _M), lambda b: (b, 0, 0)),
            pl.BlockSpec((1, _M, 4 * _C), lambda b: (b, 0, 0)),
            pl.BlockSpec((1, _M, 4), lambda b: (b, 0, 0)),
        ],
        out_specs=[
            pl.BlockSpec((1, 4, 128), lambda b: (b, 0, 0)),
            pl.BlockSpec((1, 1, 128), lambda b: (b, 0, 0)),
            pl.BlockSpec((1, 1, 128), lambda b: (b, 0, 0)),
        ],
        out_shape=[
            jax.ShapeDtypeStruct((_B, 4, 128), jnp.float32),
            jax.ShapeDtypeStruct((_B, 1, 128), jnp.float32),
            jax.ShapeDtypeStruct((_B, 1, 128), jnp.int32),
        ],
        scratch_shapes=[pltpu.VMEM((_M, _M), jnp.float32)],
        interpret=interpret,
    )(sc[:, None, :], idx[:, None, :], slot[:, None, :], regrows, proprows)


def _impl(class_logits, box_regression, proposals, interpret=False):
    lT = class_logits.T.reshape(_C, _KCH, _SB, _LN)
    rT = box_regression.T.reshape(4 * _C, _KCH, _SB, _LN)
    pT = jnp.transpose(proposals, (2, 0, 1)).reshape(4, _KCH, _SB, _LN)

    (rmax,) = _row_scores(lT, rT, pT, interpret)
    rmax = rmax.reshape(_B, _N)

    _, rtopi = lax.top_k(rmax, _M)
    rows_sorted = jnp.sort(rtopi, axis=1)
    rows_global = rows_sorted + (
        jnp.arange(_B, dtype=jnp.int32) * _N)[:, None]
    sub, regrows, proprows = _gather_rows(
        rows_global, lT, rT, pT, interpret)
    flat2 = sub.reshape(_B, _M * _C1)
    top_s, pos = lax.top_k(flat2, _TOPK)
    prow = jnp.take_along_axis(rows_sorted, pos // _C1, axis=1)
    top_i = prow * _C1 + pos % _C1

    pad = _M - _TOPK
    padi = (_NC + lax.broadcasted_iota(jnp.int32, (_B, pad), 1))
    pads = jnp.full((_B, pad), 2 * _M, jnp.int32)
    sc_p = jnp.concatenate([top_s, jnp.full((_B, pad), -2e9, jnp.float32)], axis=1)
    idx_p = jnp.concatenate([top_i, padi], axis=1)
    slot_p = jnp.concatenate([pos // _C1, pads], axis=1)

    ob, os_, ol = _nms_kernel(sc_p, idx_p, slot_p, regrows, proprows, interpret)
    boxes = ob.transpose(0, 2, 1)[:, :_DET, :]
    return boxes, os_[:, 0, :_DET], ol[:, 0, :_DET]


def kernel(class_logits, box_regression, proposals):
    return _impl(class_logits, box_regression, proposals)


# speedup vs baseline: 1.0577x; 1.0577x over previous
import math

import jax
import jax.numpy as jnp
from jax import lax
from jax.experimental import pallas as pl
from jax.experimental.pallas import tpu as pltpu

_B, _N, _C = 2, 20000, 91
_C1 = _C - 1
_NC = _N * _C1
_SCORE_THRESH = 0.05
_NMS_THRESH = 0.5
_DET = 100
_TOPK = 1000
_M = 1024
_CH = 128
_NCHUNK = _M // _CH
_IMG_W = 800.0
_IMG_H = 800.0
_CLIP = math.log(1000.0 / 16.0)
_NEG = -1e9

_R = 1000


def _k1_body(logits_ref, d_ref, prop_ref, out_ref, rmax_ref):
    l = logits_ref[0]
    m = jnp.max(l, axis=-1, keepdims=True)
    e = jnp.exp(l - m)
    s = jnp.sum(e, axis=-1, keepdims=True)
    scores = (e / s)[:, 1:]

    d = d_ref[0]
    ii = lax.broadcasted_iota(jnp.int32, (4 * _C, _C), 0)
    jj = lax.broadcasted_iota(jnp.int32, (4 * _C, _C), 1)

    def comp(k):
        sm = (ii == 4 * jj + k).astype(jnp.float32)
        return lax.dot(d, sm, precision=lax.Precision.HIGHEST)[:, 1:]

    dx = comp(0) / 10.0
    dy = comp(1) / 10.0
    dw = jnp.minimum(comp(2) / 5.0, _CLIP)
    dh = jnp.minimum(comp(3) / 5.0, _CLIP)

    p = prop_ref[0]
    px1 = p[:, 0:1]
    py1 = p[:, 1:2]
    px2 = p[:, 2:3]
    py2 = p[:, 3:4]
    w = px2 - px1
    h = py2 - py1
    cx = px1 + 0.5 * w
    cy = py1 + 0.5 * h

    pcx = dx * w + cx
    pcy = dy * h + cy
    pw = jnp.exp(dw) * w
    ph = jnp.exp(dh) * h
    x1 = jnp.clip(pcx - 0.5 * pw, 0.0, _IMG_W)
    y1 = jnp.clip(pcy - 0.5 * ph, 0.0, _IMG_H)
    x2 = jnp.clip(pcx + 0.5 * pw, 0.0, _IMG_W)
    y2 = jnp.clip(pcy + 0.5 * ph, 0.0, _IMG_H)

    valid = (scores > _SCORE_THRESH) & ((x2 - x1) >= 0.01) & ((y2 - y1) >= 0.01)
    masked = jnp.where(valid, scores, _NEG)
    out_ref[0] = masked
    rmax_ref[0, 0] = jnp.max(masked, axis=-1)


def _masked_scores(logits, d4, props, interpret=False):
    grid = (_B, _N // _R)
    return pl.pallas_call(
        _k1_body,
        grid=grid,
        in_specs=[
            pl.BlockSpec((1, _R, _C), lambda b, i: (b, i, 0)),
            pl.BlockSpec((1, _R, 4 * _C), lambda b, i: (b, i, 0)),
            pl.BlockSpec((1, _R, 4), lambda b, i: (b, i, 0)),
        ],
        out_specs=[
            pl.BlockSpec((1, _R, _C1), lambda b, i: (b, i, 0)),
            pl.BlockSpec((1, 1, _R), lambda b, i: (b * (_N // _R) + i, 0, 0)),
        ],
        out_shape=[
            jax.ShapeDtypeStruct((_B, _N, _C1), jnp.float32),
            jax.ShapeDtypeStruct((_B * (_N // _R), 1, _R), jnp.float32),
        ],
        interpret=interpret,
    )(logits, d4, props)


_RC = 2000


def _k2_body(rows_ref, m_ref, r_ref, p_ref, sub_ref, regr_ref, propr_ref):
    i = pl.program_id(1)
    rows = rows_ref[0, 0]
    colbase = i * _RC
    cols = lax.broadcasted_iota(jnp.int32, (_M, _RC), 1) + colbase
    oh = (rows[:, None] == cols).astype(jnp.float32)
    hp = lax.Precision.HIGHEST
    subc = lax.dot(oh, m_ref[0], precision=hp)
    regc = lax.dot(oh, r_ref[0], precision=hp)
    propc = lax.dot(oh, p_ref[0], precision=hp)

    @pl.when(i == 0)
    def _():
        sub_ref[0] = subc
        regr_ref[0] = regc
        propr_ref[0] = propc

    @pl.when(i > 0)
    def _():
        sub_ref[0] += subc
        regr_ref[0] += regc
        propr_ref[0] += propc


def _gather_rows(rows_sorted, masked, reg, props, interpret=False):
    return pl.pallas_call(
        _k2_body,
        grid=(_B, _N // _RC),
        in_specs=[
            pl.BlockSpec((1, 1, _M), lambda b, i: (b, 0, 0)),
            pl.BlockSpec((1, _RC, _C1), lambda b, i: (b, i, 0)),
            pl.BlockSpec((1, _RC, 4 * _C), lambda b, i: (b, i, 0)),
            pl.BlockSpec((1, _RC, 4), lambda b, i: (b, i, 0)),
        ],
        out_specs=[
            pl.BlockSpec((1, _M, _C1), lambda b, i: (b, 0, 0)),
            pl.BlockSpec((1, _M, 4 * _C), lambda b, i: (b, 0, 0)),
            pl.BlockSpec((1, _M, 4), lambda b, i: (b, 0, 0)),
        ],
        out_shape=[
            jax.ShapeDtypeStruct((_B, _M, _C1), jnp.float32),
            jax.ShapeDtypeStruct((_B, _M, 4 * _C), jnp.float32),
            jax.ShapeDtypeStruct((_B, _M, 4), jnp.float32),
        ],
        interpret=interpret,
    )(rows_sorted[:, None, :], masked, reg, props)


def _k3_body(sc_ref, idx_ref, slot_ref, regr_ref, propr_ref,
             ob_ref, os_ref, ol_ref, adj_ref):
    s = sc_ref[0, 0]
    idx = idx_ref[0, 0]
    idxf = idx.astype(jnp.float32)
    slot = slot_ref[0, 0]

    hp = lax.Precision.HIGHEST
    slots = lax.broadcasted_iota(jnp.int32, (_M, _M), 1)
    ohr = (slot[:, None] == slots).astype(jnp.float32)
    rsel = lax.dot(ohr, regr_ref[0], precision=hp)
    psel = lax.dot(ohr, propr_ref[0], precision=hp)

    rparts = []
    for c in range(_NCHUNK):
        sl = slice(c * _CH, (c + 1) * _CH)
        src = s[sl][:, None]
        idc = idxf[sl][:, None]
        gt = (s[None, :] > src) | ((s[None, :] == src) & (idxf[None, :] < idc))
        rparts.append(jnp.sum(gt.astype(jnp.int32), axis=1))
    r = jnp.concatenate(rparts)
    top = r < _TOPK

    cls = idx % _C1 + 1

    ii = lax.broadcasted_iota(jnp.int32, (4 * _C, _C), 0)
    jj = lax.broadcasted_iota(jnp.int32, (4 * _C, _C), 1)
    clsoh = (cls[:, None] == lax.broadcasted_iota(jnp.int32, (_M, _C), 1)
             ).astype(jnp.float32)

    def rcomp(k):
        sm = (ii == 4 * jj + k).astype(jnp.float32)
        tmp = lax.dot(rsel, sm, precision=hp)
        return jnp.sum(tmp * clsoh, axis=1)

    px1 = psel[:, 0]
    py1 = psel[:, 1]
    px2 = psel[:, 2]
    py2 = psel[:, 3]
    w = px2 - px1
    h = py2 - py1
    cx = px1 + 0.5 * w
    cy = py1 + 0.5 * h
    dx = rcomp(0) / 10.0
    dy = rcomp(1) / 10.0
    dw = jnp.minimum(rcomp(2) / 5.0, _CLIP)
    dh = jnp.minimum(rcomp(3) / 5.0, _CLIP)
    pcx = dx * w + cx
    pcy = dy * h + cy
    pw = jnp.exp(dw) * w
    ph = jnp.exp(dh) * h
    x1 = jnp.clip(pcx - 0.5 * pw, 0.0, _IMG_W)
    y1 = jnp.clip(pcy - 0.5 * ph, 0.0, _IMG_H)
    x2 = jnp.clip(pcx + 0.5 * pw, 0.0, _IMG_W)
    y2 = jnp.clip(pcy + 0.5 * ph, 0.0, _IMG_H)

    offs = cls.astype(jnp.float32) * 4096.0
    bx1 = x1 + offs
    by1 = y1 + offs
    bx2 = x2 + offs
    by2 = y2 + offs
    area = (bx2 - bx1) * (by2 - by1)

    topf = top.astype(jnp.float32)
    for c in range(_NCHUNK):
        sl = slice(c * _CH, (c + 1) * _CH)
        ltx = jnp.maximum(bx1[sl][:, None], bx1[None, :])
        lty = jnp.maximum(by1[sl][:, None], by1[None, :])
        rbx = jnp.minimum(bx2[sl][:, None], bx2[None, :])
        rby = jnp.minimum(by2[sl][:, None], by2[None, :])
        inter = jnp.clip(rbx - ltx, 0.0, None) * jnp.clip(rby - lty, 0.0, None)
        union = area[sl][:, None] + area[None, :] - inter
        iou = inter / jnp.maximum(union, 1e-9)
        adjc = ((iou > _NMS_THRESH) & (r[sl][:, None] < r[None, :])).astype(jnp.float32)
        adj_ref[sl, :] = adjc * topf[sl][:, None] * topf[None, :]

    def cond(carry):
        _, changed, it = carry
        return (changed > 0) & (it < _M + 1)

    def body(carry):
        keepf, _, it = carry
        sup = jnp.zeros((_M,), jnp.float32)
        for c in range(_NCHUNK):
            sl = slice(c * _CH, (c + 1) * _CH)
            sup = jnp.maximum(sup, jnp.max(keepf[sl][:, None] * adj_ref[sl, :], axis=0))
        nk = topf * (1.0 - jnp.minimum(sup, 1.0))
        ch = jnp.sum(jnp.abs(nk - keepf)).astype(jnp.int32)
        return nk, ch, it + 1

    keepf, _, _ = lax.while_loop(cond, body, (topf, jnp.int32(1), jnp.int32(0)))

    condf = keepf * (s > -1e8).astype(jnp.float32)
    cond_b = condf > 0
    uf = topf * (1.0 - condf)
    kb = jnp.zeros((_M,), jnp.float32)
    ub = jnp.zeros((_M,), jnp.float32)
    for c in range(_NCHUNK):
        sl = slice(c * _CH, (c + 1) * _CH)
        rlt = (r[sl][:, None] < r[None, :]).astype(jnp.float32)
        kb = kb + jnp.sum(condf[sl][:, None] * rlt, axis=0)
        ub = ub + jnp.sum(uf[sl][:, None] * rlt, axis=0)
    nktot = jnp.sum(condf)
    frank = jnp.where(cond_b, kb, nktot + ub).astype(jnp.int32)
    frank = jnp.where(top, frank, jnp.int32(100000))

    krow = lax.broadcasted_iota(jnp.int32, (128, _M), 0)
    oh2 = (frank[None, :] == krow).astype(jnp.float32)

    def sel(x):
        return jnp.sum(oh2 * x[None, :], axis=1)

    fsc = jnp.where(cond_b, s, _NEG)
    os_ref[0, 0] = sel(fsc)
    ol_ref[0, 0] = sel(cls.astype(jnp.float32)).astype(jnp.int32)
    ob_ref[0] = jnp.stack([sel(x1), sel(y1), sel(x2), sel(y2)], axis=0)


def _nms_kernel(sc, idx, slot, regrows, proprows, interpret=False):
    return pl.pallas_call(
        _k3_body,
        grid=(_B,),
        in_specs=[
            pl.BlockSpec((1, 1, _M), lambda b: (b, 0, 0)),
            pl.BlockSpec((1, 1, _M), lambda b: (b, 0, 0)),
            pl.BlockSpec((1, 1, _M), lambda b: (b, 0, 0)),
            pl.BlockSpec((1, _M, 4 * _C), lambda b: (b, 0, 0)),
            pl.BlockSpec((1, _M, 4), lambda b: (b, 0, 0)),
        ],
        out_specs=[
            pl.BlockSpec((1, 4, 128), lambda b: (b, 0, 0)),
            pl.BlockSpec((1, 1, 128), lambda b: (b, 0, 0)),
            pl.BlockSpec((1, 1, 128), lambda b: (b, 0, 0)),
        ],
        out_shape=[
            jax.ShapeDtypeStruct((_B, 4, 128), jnp.float32),
            jax.ShapeDtypeStruct((_B, 1, 128), jnp.float32),
            jax.ShapeDtypeStruct((_B, 1, 128), jnp.int32),
        ],
        scratch_shapes=[pltpu.VMEM((_M, _M), jnp.float32)],
        interpret=interpret,
    )(sc[:, None, :], idx[:, None, :], slot[:, None, :], regrows, proprows)


def _impl(class_logits, box_regression, proposals, interpret=False):
    logits = class_logits.reshape(_B, _N, _C)
    d4 = box_regression.reshape(_B, _N, 4 * _C)

    masked, rmax = _masked_scores(logits, d4, proposals, interpret)
    rmax = rmax.reshape(_B, _N)

    _, rtopi = lax.top_k(rmax, _M)
    rows_sorted = jnp.sort(rtopi, axis=1)
    sub, regrows, proprows = _gather_rows(
        rows_sorted, masked, d4, proposals, interpret)
    flat2 = sub.reshape(_B, _M * _C1)
    top_s, pos = lax.top_k(flat2, _TOPK)
    prow = jnp.take_along_axis(rows_sorted, pos // _C1, axis=1)
    top_i = prow * _C1 + pos % _C1

    pad = _M - _TOPK
    padi = (_NC + lax.broadcasted_iota(jnp.int32, (_B, pad), 1))
    pads = jnp.full((_B, pad), 2 * _M, jnp.int32)
    sc_p = jnp.concatenate([top_s, jnp.full((_B, pad), -2e9, jnp.float32)], axis=1)
    idx_p = jnp.concatenate([top_i, padi], axis=1)
    slot_p = jnp.concatenate([pos // _C1, pads], axis=1)

    ob, os_, ol = _nms_kernel(sc_p, idx_p, slot_p, regrows, proprows, interpret)
    boxes = ob.transpose(0, 2, 1)[:, :_DET, :]
    return boxes, os_[:, 0, :_DET], ol[:, 0, :_DET]


def kernel(class_logits, box_regression, proposals):
    return _impl(class_logits, box_regression, proposals)
